# baseline (speedup 1.0000x reference)
import functools
import math

import jax
import jax.numpy as jnp
from jax import lax
from jax.experimental import pallas as pl
from jax.experimental.pallas import tpu as pltpu
from jax.experimental.pallas import tpu_sc as plsc

N = 4096
V = 32000
PAD = 0
SMOOTH = 0.1
CONF = 1.0 - SMOOTH
EPS = SMOOTH / (V - 2)
ROW_CONST = CONF * math.log(CONF) + (V - 2) * EPS * math.log(EPS)

NC = 2
NS = 16
L = 16
NW = NC * NS

ROWS_SC = 1024
ROWS_TC = N - ROWS_SC
RPW = ROWS_SC // NW
IDX_BATCH = 128
SPLAT_BATCHES = RPW * L // IDX_BATCH

UNROLL = 40
SLICES = V // L

BR = 128
NRB = ROWS_TC // BR


def _sc_part(x, y):
    mesh = plsc.VectorSubcoreMesh(core_axis_name="c", subcore_axis_name="s")

    @functools.partial(
        pl.kernel,
        mesh=mesh,
        out_type=[
            jax.ShapeDtypeStruct((NW, L), jnp.float32),
            jax.ShapeDtypeStruct((NW, L), jnp.float32),
        ],
        scratch_types=[
            pltpu.VMEM((RPW * L,), jnp.int32),
            pltpu.VMEM((RPW * L,), jnp.int32),
            pltpu.VMEM((2, V), jnp.float32),
            pltpu.VMEM((L,), jnp.float32),
            pltpu.VMEM((L,), jnp.float32),
            pltpu.SemaphoreType.DMA,
            pltpu.SemaphoreType.DMA,
            pltpu.SemaphoreType.DMA,
        ],
    )
    def k(x_hbm, y_hbm, p_hbm, k_hbm,
          idxs_v, yspl_v, rowbuf, p_v, k_v,
          sem, sem0, sem1):
        wid = lax.axis_index("s") * NC + lax.axis_index("c")
        iv = lax.iota(jnp.int32, L)
        row0 = ROWS_TC + wid * RPW

        zi = jnp.zeros((L,), jnp.int32)
        for j in range(RPW):
            idxs_v[pl.ds(j * L, L)] = zi + (row0 + j)
        for j in range(SPLAT_BATCHES):
            pltpu.async_copy(
                y_hbm.at[idxs_v.at[pl.ds(j * IDX_BATCH, IDX_BATCH)]],
                yspl_v.at[pl.ds(j * IDX_BATCH, IDX_BATCH)], sem).wait()

        sems = (sem0, sem1)
        cps = [pltpu.async_copy(x_hbm.at[row0], rowbuf.at[0], sem0), None]

        pacc = jnp.zeros((L,), jnp.float32)
        kacc = jnp.zeros((L,), jnp.float32)
        zero = jnp.zeros((L,), jnp.float32)
        fone = jnp.ones((L,), jnp.float32)

        for r in range(RPW):
            if r + 1 < RPW:
                cps[(r + 1) % 2] = pltpu.async_copy(
                    x_hbm.at[row0 + r + 1], rowbuf.at[(r + 1) % 2],
                    sems[(r + 1) % 2])
            cps[r % 2].wait()
            buf = rowbuf.at[r % 2]
            yspl = yspl_v[pl.ds(r * L, L)]

            @plsc.parallel_loop(0, V, step=UNROLL * L, unroll=2,
                                carry=(zero, zero, zero, zero, zero, zero))
            def reduce_body(off, accs):
                a0, a1, a2, a3, g0, g1 = accs
                for s in range(UNROLL):
                    sl = buf[pl.ds(off + s * L, L)]
                    hit = jnp.where(off + s * L + iv == yspl, sl, 0.0)
                    if s % 2 == 0:
                        g0 = g0 + hit
                    else:
                        g1 = g1 + hit
                    if s % 4 == 0:
                        a0 = a0 + sl
                    elif s % 4 == 1:
                        a1 = a1 + sl
                    elif s % 4 == 2:
                        a2 = a2 + sl
                    else:
                        a3 = a3 + sl
                return a0, a1, a2, a3, g0, g1

            a0, a1, a2, a3, g0, g1 = reduce_body
            tot = (a0 + a1) + (a2 + a3)
            c0vec = jnp.where(iv == 0, buf[pl.ds(0, L)], 0.0)
            mf = jnp.where(yspl != PAD, fone, zero)
            prow = (jnp.float32(CONF - EPS) * (g0 + g1)
                    + jnp.float32(EPS) * (tot - c0vec))
            pacc = pacc + mf * prow
            kacc = kacc + jnp.where(iv == 0, mf, zero)

        p_v[...] = pacc
        k_v[...] = kacc
        pltpu.sync_copy(p_v, p_hbm.at[wid])
        pltpu.sync_copy(k_v, k_hbm.at[wid])

    return k(x, y)


def _tc_body(y_ref, x_ref, p_ref, k_ref):
    blk = x_ref[...]
    yv = y_ref[0]
    mask = yv != PAD
    col = lax.broadcasted_iota(jnp.int32, (BR, V), 1)
    w = jnp.where(col == yv, jnp.float32(CONF), jnp.float32(EPS))
    prow = jnp.sum(w * blk, axis=1, keepdims=True) - jnp.float32(EPS) * blk[:, 0:1]
    p = jnp.sum(jnp.where(mask, prow, 0.0))
    cnt = jnp.sum(jnp.where(mask, 1.0, 0.0))
    r = pl.program_id(0)

    @pl.when(r == 0)
    def _():
        p_ref[0, 0] = p
        k_ref[0, 0] = cnt

    @pl.when(r != 0)
    def _():
        p_ref[0, 0] = p_ref[0, 0] + p
        k_ref[0, 0] = k_ref[0, 0] + cnt


def _tc_part(x, y3):
    return pl.pallas_call(
        _tc_body,
        grid=(NRB,),
        in_specs=[
            pl.BlockSpec((1, BR, 1), lambda r: (r, 0, 0)),
            pl.BlockSpec((BR, V), lambda r: (r, 0)),
        ],
        out_specs=[
            pl.BlockSpec(memory_space=pltpu.SMEM),
            pl.BlockSpec(memory_space=pltpu.SMEM),
        ],
        out_shape=[
            jax.ShapeDtypeStruct((1, 1), jnp.float32),
            jax.ShapeDtypeStruct((1, 1), jnp.float32),
        ],
    )(y3, x)


def kernel(x, y):
    p_sc, k_sc = _sc_part(x, y)
    p_tc, k_tc = _tc_part(x, y.reshape(N // BR, BR, 1))
    p = p_tc[0, 0] + jnp.sum(p_sc)
    cnt = k_tc[0, 0] + jnp.sum(k_sc)
    return cnt * jnp.float32(ROW_CONST) - p


# speedup vs baseline: 1.2085x; 1.2085x over previous
import functools
import math

import jax
import jax.numpy as jnp
from jax import lax
from jax.experimental import pallas as pl
from jax.experimental.pallas import tpu as pltpu
from jax.experimental.pallas import tpu_sc as plsc

N = 4096
V = 32000
PAD = 0
SMOOTH = 0.1
CONF = 1.0 - SMOOTH
EPS = SMOOTH / (V - 2)
ROW_CONST = CONF * math.log(CONF) + (V - 2) * EPS * math.log(EPS)

NC = 2
NS = 16
L = 16
NW = NC * NS

ROWS_SC = 1024
ROWS_TC = N - ROWS_SC
RPW = ROWS_SC // NW
IDX_BATCH = 128
SPLAT_BATCHES = RPW * L // IDX_BATCH

UNROLL = 25
SLICES = V // L

BR = 128
NRB = ROWS_TC // BR


def _sc_part(x, y):
    mesh = plsc.VectorSubcoreMesh(core_axis_name="c", subcore_axis_name="s")

    @functools.partial(
        pl.kernel,
        mesh=mesh,
        out_type=[
            jax.ShapeDtypeStruct((NW, L), jnp.float32),
            jax.ShapeDtypeStruct((NW, L), jnp.float32),
        ],
        scratch_types=[
            pltpu.VMEM((RPW * L,), jnp.int32),
            pltpu.VMEM((RPW * L,), jnp.int32),
            pltpu.VMEM((2, V), jnp.float32),
            pltpu.VMEM((L,), jnp.float32),
            pltpu.VMEM((L,), jnp.float32),
            pltpu.SemaphoreType.DMA,
            pltpu.SemaphoreType.DMA,
            pltpu.SemaphoreType.DMA,
        ],
    )
    def k(x_hbm, y_hbm, p_hbm, k_hbm,
          idxs_v, yspl_v, rowbuf, p_v, k_v,
          sem, sem0, sem1):
        wid = lax.axis_index("s") * NC + lax.axis_index("c")
        iv = lax.iota(jnp.int32, L)
        row0 = ROWS_TC + wid * RPW

        zi = jnp.zeros((L,), jnp.int32)
        for j in range(RPW):
            idxs_v[pl.ds(j * L, L)] = zi + (row0 + j)
        for j in range(SPLAT_BATCHES):
            pltpu.async_copy(
                y_hbm.at[idxs_v.at[pl.ds(j * IDX_BATCH, IDX_BATCH)]],
                yspl_v.at[pl.ds(j * IDX_BATCH, IDX_BATCH)], sem).wait()

        sems = (sem0, sem1)
        cps = [pltpu.async_copy(x_hbm.at[row0], rowbuf.at[0], sem0), None]

        pacc = jnp.zeros((L,), jnp.float32)
        kacc = jnp.zeros((L,), jnp.float32)
        zero = jnp.zeros((L,), jnp.float32)
        fone = jnp.ones((L,), jnp.float32)

        for r in range(RPW):
            if r + 1 < RPW:
                cps[(r + 1) % 2] = pltpu.async_copy(
                    x_hbm.at[row0 + r + 1], rowbuf.at[(r + 1) % 2],
                    sems[(r + 1) % 2])
            cps[r % 2].wait()
            buf = rowbuf.at[r % 2]
            yspl = yspl_v[pl.ds(r * L, L)]

            @plsc.parallel_loop(0, V, step=UNROLL * L, unroll=2,
                                carry=(zero, zero, zero, zero, zero, zero))
            def reduce_body(off, accs):
                a0, a1, a2, a3, g0, g1 = accs
                for s in range(UNROLL):
                    sl = buf[pl.ds(off + s * L, L)]
                    hit = jnp.where(off + s * L + iv == yspl, sl, 0.0)
                    if s % 2 == 0:
                        g0 = g0 + hit
                    else:
                        g1 = g1 + hit
                    if s % 4 == 0:
                        a0 = a0 + sl
                    elif s % 4 == 1:
                        a1 = a1 + sl
                    elif s % 4 == 2:
                        a2 = a2 + sl
                    else:
                        a3 = a3 + sl
                return a0, a1, a2, a3, g0, g1

            a0, a1, a2, a3, g0, g1 = reduce_body
            tot = (a0 + a1) + (a2 + a3)
            c0vec = jnp.where(iv == 0, buf[pl.ds(0, L)], 0.0)
            mf = jnp.where(yspl != PAD, fone, zero)
            prow = (jnp.float32(CONF - EPS) * (g0 + g1)
                    + jnp.float32(EPS) * (tot - c0vec))
            pacc = pacc + mf * prow
            kacc = kacc + jnp.where(iv == 0, mf, zero)

        p_v[...] = pacc
        k_v[...] = kacc
        pltpu.sync_copy(p_v, p_hbm.at[wid])
        pltpu.sync_copy(k_v, k_hbm.at[wid])

    return k(x, y)


def _tc_body(y_ref, x_ref, p_ref, k_ref):
    blk = x_ref[...]
    yv = y_ref[0]
    mask = yv != PAD
    col = lax.broadcasted_iota(jnp.int32, (BR, V), 1)
    w = jnp.where(col == yv, jnp.float32(CONF), jnp.float32(EPS))
    prow = jnp.sum(w * blk, axis=1, keepdims=True) - jnp.float32(EPS) * blk[:, 0:1]
    p = jnp.sum(jnp.where(mask, prow, 0.0))
    cnt = jnp.sum(jnp.where(mask, 1.0, 0.0))
    r = pl.program_id(0)

    @pl.when(r == 0)
    def _():
        p_ref[0, 0] = p
        k_ref[0, 0] = cnt

    @pl.when(r != 0)
    def _():
        p_ref[0, 0] = p_ref[0, 0] + p
        k_ref[0, 0] = k_ref[0, 0] + cnt


def _tc_part(x, y3):
    return pl.pallas_call(
        _tc_body,
        grid=(NRB,),
        in_specs=[
            pl.BlockSpec((1, BR, 1), lambda r: (r, 0, 0)),
            pl.BlockSpec((BR, V), lambda r: (r, 0)),
        ],
        out_specs=[
            pl.BlockSpec(memory_space=pltpu.SMEM),
            pl.BlockSpec(memory_space=pltpu.SMEM),
        ],
        out_shape=[
            jax.ShapeDtypeStruct((1, 1), jnp.float32),
            jax.ShapeDtypeStruct((1, 1), jnp.float32),
        ],
    )(y3, x)


def kernel(x, y):
    p_sc, k_sc = _sc_part(x, y)
    p_tc, k_tc = _tc_part(x, y.reshape(N // BR, BR, 1))
    p = p_tc[0, 0] + jnp.sum(p_sc)
    cnt = k_tc[0, 0] + jnp.sum(k_sc)
    return cnt * jnp.float32(ROW_CONST) - p
